# baseline (speedup 1.0000x reference)
import functools
import math

import jax
import jax.numpy as jnp
from jax import lax
from jax.experimental import pallas as pl
from jax.experimental.pallas import tpu as pltpu
from jax.experimental.pallas import tpu_sc as plsc

N_HEADS = 16
N_KV_HEADS = 4
HEAD_SIZE = 128
MAX_SEQLEN = 2048
SPARSE_BLOCK = 64
VLLM_BLOCK = 16
LOCAL_BLOCKS = 8
VERT_STRIDE = 4
NUM_SEQS = 32
BLOCKS_PER_SEQ = MAX_SEQLEN // VLLM_BLOCK
NUM_SPARSE_BLOCKS = MAX_SEQLEN // SPARSE_BLOCK
VPB = SPARSE_BLOCK // VLLM_BLOCK
MAX_SLOTS = 14
NUM_VB = MAX_SLOTS * VPB
T_PACK = MAX_SLOTS * SPARSE_BLOCK
NUM_CACHE_ROWS = NUM_SEQS * BLOCKS_PER_SEQ * N_KV_HEADS
SM_SCALE = 1.0 / math.sqrt(HEAD_SIZE)
NEG_INF = -1e30


def _attn_body(q_ref, k_ref, v_ref, b_ref, o_ref):
    q4 = q_ref[0].reshape(N_KV_HEADS, N_HEADS // N_KV_HEADS, HEAD_SIZE)
    kc = k_ref[0]
    vc = v_ref[0]
    bias = b_ref[0]

    sc = jax.lax.dot_general(
        q4, kc,
        dimension_numbers=(((2,), (2,)), ((0,), (0,))),
        preferred_element_type=jnp.float32,
    ).reshape(N_HEADS, T_PACK) * SM_SCALE + bias

    m = jnp.max(sc, axis=-1, keepdims=True)
    p = jnp.exp(sc - m)
    l = jnp.sum(p, axis=-1, keepdims=True)

    pv = jax.lax.dot_general(
        p.reshape(N_KV_HEADS, N_HEADS // N_KV_HEADS, T_PACK), vc,
        dimension_numbers=(((2,), (1,)), ((0,), (0,))),
        preferred_element_type=jnp.float32,
    ).reshape(N_HEADS, HEAD_SIZE)
    o_ref[0] = pv / l


def _routing(block_tables, context_lens):
    ctx = context_lens.astype(jnp.int32)
    qblk = (ctx - 1) // SPARSE_BLOCK
    j = jnp.arange(NUM_SPARSE_BLOCKS, dtype=jnp.int32)
    vis = (j[None, :] <= qblk[:, None]) & (
        (qblk[:, None] - j[None, :] < LOCAL_BLOCKS)
        | ((j[None, :] + 1) % VERT_STRIDE == 0))
    key = jnp.where(vis, j[None, :], jnp.int32(10_000))
    packed = jnp.sort(key, axis=1)[:, :MAX_SLOTS]
    counts = jnp.sum(vis.astype(jnp.int32), axis=1)
    slot = jnp.arange(MAX_SLOTS, dtype=jnp.int32)
    valid = slot[None, :] < counts[:, None]
    visj = jnp.where(valid, packed, qblk[:, None])
    lim = jnp.where(
        valid,
        jnp.clip(ctx[:, None] - SPARSE_BLOCK * visj, 0, SPARSE_BLOCK),
        0).astype(jnp.int32)

    vb = (VPB * visj[:, :, None]
          + jnp.arange(VPB, dtype=jnp.int32)[None, None, :]).reshape(
              NUM_SEQS, NUM_VB)
    cb = jnp.take_along_axis(block_tables, vb, axis=1)

    t_in = jnp.arange(SPARSE_BLOCK, dtype=jnp.int32)
    bias = jnp.where(t_in[None, None, :] < lim[:, :, None], 0.0,
                     NEG_INF).reshape(NUM_SEQS, 1, T_PACK).astype(jnp.float32)
    return cb, bias


def _sc_gather_transpose(k, v, cb):
    mesh = plsc.VectorSubcoreMesh(core_axis_name="c", subcore_axis_name="s")
    out_sds = jax.ShapeDtypeStruct(
        (NUM_SEQS, N_KV_HEADS, T_PACK, HEAD_SIZE), jnp.float32)
    @functools.partial(
        pl.kernel, out_type=[out_sds, out_sds], mesh=mesh,
        scratch_types=[
            pltpu.VMEM((NUM_VB,), jnp.int32),
            pltpu.VMEM((2, 1, N_KV_HEADS, HEAD_SIZE, VLLM_BLOCK), jnp.float32),
            pltpu.VMEM((2, 1, N_KV_HEADS, HEAD_SIZE, VLLM_BLOCK), jnp.float32),
            pltpu.VMEM((2, N_KV_HEADS, VLLM_BLOCK, HEAD_SIZE), jnp.float32),
            pltpu.VMEM((2, N_KV_HEADS, VLLM_BLOCK, HEAD_SIZE), jnp.float32),
            pltpu.SemaphoreType.DMA,
            pltpu.SemaphoreType.DMA,
            pltpu.SemaphoreType.DMA,
            pltpu.SemaphoreType.DMA,
        ],
    )
    def gt(k_hbm, v_hbm, cb_hbm, ko_hbm, vo_hbm, cbv, kin, vin, kout, vout,
           sk, sv, sko, svo):
        s = lax.axis_index("s") * 2 + lax.axis_index("c")
        iota16 = lax.iota(jnp.int32, VLLM_BLOCK)
        zeros16 = iota16 - iota16
        pltpu.sync_copy(cb_hbm.at[s], cbv)

        def gathers(c, b):
            return (
                pltpu.make_async_copy(
                    k_hbm.at[cbv.at[pl.ds(c, 1)]], kin.at[b], sk),
                pltpu.make_async_copy(
                    v_hbm.at[cbv.at[pl.ds(c, 1)]], vin.at[b], sv),
            )

        def out_copies(c, b):
            tok = c * VLLM_BLOCK
            cps = []
            for g in range(N_KV_HEADS):
                cps.append(pltpu.make_async_copy(
                    kout.at[b, g],
                    ko_hbm.at[s, g, pl.ds(tok, VLLM_BLOCK), :], sko))
                cps.append(pltpu.make_async_copy(
                    vout.at[b, g],
                    vo_hbm.at[s, g, pl.ds(tok, VLLM_BLOCK), :], svo))
            return cps

        for cp in gathers(0, 0):
            cp.start()

        def body(c, carry):
            b = lax.rem(c, 2)

            for cp in gathers(c, b):
                cp.wait()

            @pl.when(c + 1 < NUM_VB)
            def _():
                for cp in gathers(c + 1, 1 - b):
                    cp.start()

            @pl.when(c >= 2)
            def _():
                for cp in out_copies(c - 2, b):
                    cp.wait()

            def transpose(src, dst):
                for g in range(N_KV_HEADS):

                    def dgrp(i, _):
                        for dd in range(VLLM_BLOCK):
                            d = i * VLLM_BLOCK + dd
                            val = src[b, 0, g, d, :]
                            plsc.store_scatter(
                                dst, [zeros16 + b, zeros16 + g, iota16,
                                      zeros16 + d], val)
                        return 0

                    lax.fori_loop(0, HEAD_SIZE // VLLM_BLOCK, dgrp, 0)

            transpose(kin, kout)
            transpose(vin, vout)

            for cp in out_copies(c, b):
                cp.start()
            return 0

        lax.fori_loop(0, NUM_VB, body, 0)
        for cp in out_copies(NUM_VB - 2, 0):
            cp.wait()
        for cp in out_copies(NUM_VB - 1, 1):
            cp.wait()

    return gt(k, v, cb)


@jax.jit
def kernel(q, k, v, block_tables, context_lens):
    cb, bias = _routing(block_tables, context_lens)

    g_off = jnp.arange(N_KV_HEADS, dtype=jnp.int32)
    cb2 = (N_KV_HEADS * cb[:, None, :]
           + g_off[None, :, None]).reshape(-1)

    n_chunk = 2
    cs = NUM_SEQS // n_chunk
    cb2c = cb2.reshape(n_chunk, cs * N_KV_HEADS * NUM_VB)
    k2 = k.reshape(NUM_CACHE_ROWS, HEAD_SIZE, VLLM_BLOCK)
    v2 = v.reshape(NUM_CACHE_ROWS, HEAD_SIZE, VLLM_BLOCK)

    def compact(x2, c):
        g = jnp.take(x2, cb2c[c], axis=0)
        g = g.reshape(cs, N_KV_HEADS, NUM_VB, HEAD_SIZE, VLLM_BLOCK)
        g = jnp.swapaxes(g, -1, -2)
        return g.reshape(cs, N_KV_HEADS, T_PACK, HEAD_SIZE)

    grid_spec = pl.GridSpec(
        grid=(cs,),
        in_specs=[
            pl.BlockSpec((1, N_HEADS, HEAD_SIZE), lambda s: (s, 0, 0)),
            pl.BlockSpec((1, N_KV_HEADS, T_PACK, HEAD_SIZE),
                         lambda s: (s, 0, 0, 0)),
            pl.BlockSpec((1, N_KV_HEADS, T_PACK, HEAD_SIZE),
                         lambda s: (s, 0, 0, 0)),
            pl.BlockSpec((1, 1, T_PACK), lambda s: (s, 0, 0)),
        ],
        out_specs=pl.BlockSpec((1, N_HEADS, HEAD_SIZE), lambda s: (s, 0, 0)),
    )

    outs = []
    for c in range(n_chunk):
        kc = compact(k2, c)
        vc = compact(v2, c)
        outs.append(pl.pallas_call(
            _attn_body,
            grid_spec=grid_spec,
            out_shape=jax.ShapeDtypeStruct((cs, N_HEADS, HEAD_SIZE),
                                           jnp.float32),
            compiler_params=pltpu.CompilerParams(
                dimension_semantics=("arbitrary",)),
        )(q[c * cs:(c + 1) * cs], kc, vc, bias[c * cs:(c + 1) * cs]))
    return jnp.concatenate(outs, axis=0)


# speedup vs baseline: 1.0316x; 1.0316x over previous
import math

import jax
import jax.numpy as jnp
from jax.experimental import pallas as pl
from jax.experimental.pallas import tpu as pltpu

N_HEADS = 16
N_KV_HEADS = 4
HEAD_SIZE = 128
MAX_SEQLEN = 2048
SPARSE_BLOCK = 64
VLLM_BLOCK = 16
LOCAL_BLOCKS = 8
VERT_STRIDE = 4
NUM_SEQS = 32
BLOCKS_PER_SEQ = MAX_SEQLEN // VLLM_BLOCK
NUM_SPARSE_BLOCKS = MAX_SEQLEN // SPARSE_BLOCK
VPB = SPARSE_BLOCK // VLLM_BLOCK
MAX_SLOTS = 14
NUM_VB = MAX_SLOTS * VPB
T_PACK = MAX_SLOTS * SPARSE_BLOCK
NUM_CACHE_ROWS = NUM_SEQS * BLOCKS_PER_SEQ * N_KV_HEADS
SM_SCALE = 1.0 / math.sqrt(HEAD_SIZE)
NEG_INF = -1e30


def _attn_body(q_ref, k_ref, v_ref, b_ref, o_ref):
    q4 = q_ref[0].reshape(N_KV_HEADS, N_HEADS // N_KV_HEADS, HEAD_SIZE)
    kc = k_ref[0]
    vc = v_ref[0]
    bias = b_ref[0]

    sc = jax.lax.dot_general(
        q4, kc,
        dimension_numbers=(((2,), (2,)), ((0,), (0,))),
        preferred_element_type=jnp.float32,
    ).reshape(N_HEADS, T_PACK) * SM_SCALE + bias

    m = jnp.max(sc, axis=-1, keepdims=True)
    p = jnp.exp(sc - m)
    l = jnp.sum(p, axis=-1, keepdims=True)

    pv = jax.lax.dot_general(
        p.reshape(N_KV_HEADS, N_HEADS // N_KV_HEADS, T_PACK), vc,
        dimension_numbers=(((2,), (1,)), ((0,), (0,))),
        preferred_element_type=jnp.float32,
    ).reshape(N_HEADS, HEAD_SIZE)
    o_ref[0] = pv / l


def _routing(block_tables, context_lens):
    ctx = context_lens.astype(jnp.int32)
    qblk = (ctx - 1) // SPARSE_BLOCK
    j = jnp.arange(NUM_SPARSE_BLOCKS, dtype=jnp.int32)
    vis = (j[None, :] <= qblk[:, None]) & (
        (qblk[:, None] - j[None, :] < LOCAL_BLOCKS)
        | ((j[None, :] + 1) % VERT_STRIDE == 0))
    key = jnp.where(vis, j[None, :], jnp.int32(10_000))
    packed = jnp.sort(key, axis=1)[:, :MAX_SLOTS]
    counts = jnp.sum(vis.astype(jnp.int32), axis=1)
    slot = jnp.arange(MAX_SLOTS, dtype=jnp.int32)
    valid = slot[None, :] < counts[:, None]
    visj = jnp.where(valid, packed, qblk[:, None])
    lim = jnp.where(
        valid,
        jnp.clip(ctx[:, None] - SPARSE_BLOCK * visj, 0, SPARSE_BLOCK),
        0).astype(jnp.int32)

    vb = (VPB * visj[:, :, None]
          + jnp.arange(VPB, dtype=jnp.int32)[None, None, :]).reshape(
              NUM_SEQS, NUM_VB)
    cb = jnp.take_along_axis(block_tables, vb, axis=1)

    t_in = jnp.arange(SPARSE_BLOCK, dtype=jnp.int32)
    bias = jnp.where(t_in[None, None, :] < lim[:, :, None], 0.0,
                     NEG_INF).reshape(NUM_SEQS, 1, T_PACK).astype(jnp.float32)
    return cb, bias


@jax.jit
def kernel(q, k, v, block_tables, context_lens):
    cb, bias = _routing(block_tables, context_lens)

    g_off = jnp.arange(N_KV_HEADS, dtype=jnp.int32)
    cb2 = (N_KV_HEADS * cb[:, None, :]
           + g_off[None, :, None]).reshape(-1)

    def compact(x):
        x2 = x.reshape(NUM_CACHE_ROWS, HEAD_SIZE, VLLM_BLOCK)
        g = jnp.take(x2, cb2, axis=0)
        g = g.reshape(NUM_SEQS, N_KV_HEADS, NUM_VB, HEAD_SIZE, VLLM_BLOCK)
        g = jnp.swapaxes(g, -1, -2)
        return g.reshape(NUM_SEQS, N_KV_HEADS, T_PACK, HEAD_SIZE)

    kc = compact(k)
    vc = compact(v)

    grid_spec = pl.GridSpec(
        grid=(NUM_SEQS,),
        in_specs=[
            pl.BlockSpec((1, N_HEADS, HEAD_SIZE), lambda s: (s, 0, 0)),
            pl.BlockSpec((1, N_KV_HEADS, T_PACK, HEAD_SIZE),
                         lambda s: (s, 0, 0, 0)),
            pl.BlockSpec((1, N_KV_HEADS, T_PACK, HEAD_SIZE),
                         lambda s: (s, 0, 0, 0)),
            pl.BlockSpec((1, 1, T_PACK), lambda s: (s, 0, 0)),
        ],
        out_specs=pl.BlockSpec((1, N_HEADS, HEAD_SIZE), lambda s: (s, 0, 0)),
    )

    out = pl.pallas_call(
        _attn_body,
        grid_spec=grid_spec,
        out_shape=jax.ShapeDtypeStruct((NUM_SEQS, N_HEADS, HEAD_SIZE),
                                       jnp.float32),
        compiler_params=pltpu.CompilerParams(
            dimension_semantics=("arbitrary",)),
    )(q, kc, vc, bias)
    return out
